# Initial kernel scaffold; baseline (speedup 1.0000x reference)
import jax
import jax.numpy as jnp
from jax.experimental import pallas as pl


def kernel(input, hidden, extra_notes, enc_w, enc_lyr_w, W_ih0, W_hh0, b_ih0, b_hh0, W_ih1, W_hh1, b_ih1, b_hh1, dec_w, dec_b):
    raise NotImplementedError("write your pallas kernel here")


# speedup vs baseline: 2.7984x; 2.7984x over previous
import functools

import jax
import jax.numpy as jnp
from jax import lax
from jax.experimental import pallas as pl
from jax.experimental.pallas import tpu as pltpu
from jax.experimental.pallas import tpu_sc as plsc

_NTOKEN = 10000
_NINP = 512
_NHID = 1024
_SEQ = 64
_BATCH = 32
_NG = 3 * _NHID
_B = _SEQ * _BATCH

_NW = 32
_BPW = _B // _NW


def _make_emb_gather():
    mesh = plsc.VectorSubcoreMesh(core_axis_name="c", subcore_axis_name="s")
    out = jax.ShapeDtypeStruct((_B, _NINP), jnp.float32)

    @functools.partial(
        pl.kernel,
        out_type=[out, out],
        mesh=mesh,
        scratch_types=[
            pltpu.VMEM((_BPW,), jnp.int32),
            pltpu.VMEM((_BPW, _NINP), jnp.float32),
            pltpu.VMEM((_BPW, _NINP), jnp.float32),
            pltpu.SemaphoreType.DMA,
            pltpu.SemaphoreType.DMA,
        ],
    )
    def emb_gather(ta, ia, tb, ib, out_a, out_b, idx_v, rows_a, rows_b, sem_a, sem_b):
        wid = lax.axis_index("s") * 2 + lax.axis_index("c")
        base = wid * _BPW
        pltpu.sync_copy(ia.at[pl.ds(base, _BPW)], idx_v)
        cp_a = pltpu.async_copy(ta.at[idx_v], rows_a, sem_a)
        cp_a.wait()
        pltpu.sync_copy(rows_a, out_a.at[pl.ds(base, _BPW)])
        pltpu.sync_copy(ib.at[pl.ds(base, _BPW)], idx_v)
        cp_b = pltpu.async_copy(tb.at[idx_v], rows_b, sem_b)
        cp_b.wait()
        pltpu.sync_copy(rows_b, out_b.at[pl.ds(base, _BPW)])

    return emb_gather


_emb_gather_cache = []


def _emb_gather(ta, ia, tb, ib):
    if not _emb_gather_cache:
        _emb_gather_cache.append(_make_emb_gather())
    return _emb_gather_cache[0](ta, ia, tb, ib)


_DN = (((1,), (1,)), ((), ()))


def _proj2_body(x1_ref, x2_ref, w_ref, b_ref, o_ref):
    acc = lax.dot_general(x1_ref[...], w_ref[:, :_NINP], _DN,
                          preferred_element_type=jnp.float32)
    acc = acc + lax.dot_general(x2_ref[...], w_ref[:, _NINP:], _DN,
                                preferred_element_type=jnp.float32)
    o_ref[...] = acc + b_ref[...]


def _proj_body(x_ref, w_ref, b_ref, o_ref):
    o_ref[...] = lax.dot_general(x_ref[...], w_ref[...], _DN,
                                 preferred_element_type=jnp.float32) + b_ref[...]


_M_BLK = 256


def _input_proj0(emb, emb2, w_ih, b_ih):
    return pl.pallas_call(
        _proj2_body,
        grid=(_B // _M_BLK,),
        in_specs=[
            pl.BlockSpec((_M_BLK, _NINP), lambda i: (i, 0)),
            pl.BlockSpec((_M_BLK, _NINP), lambda i: (i, 0)),
            pl.BlockSpec((_NG, 2 * _NINP), lambda i: (0, 0)),
            pl.BlockSpec((1, _NG), lambda i: (0, 0)),
        ],
        out_specs=pl.BlockSpec((_M_BLK, _NG), lambda i: (i, 0)),
        out_shape=jax.ShapeDtypeStruct((_B, _NG), jnp.float32),
    )(emb, emb2, w_ih, b_ih.reshape(1, _NG))


def _input_proj1(x, w_ih, b_ih):
    return pl.pallas_call(
        _proj_body,
        grid=(_B // _M_BLK,),
        in_specs=[
            pl.BlockSpec((_M_BLK, _NHID), lambda i: (i, 0)),
            pl.BlockSpec((_NG, _NHID), lambda i: (0, 0)),
            pl.BlockSpec((1, _NG), lambda i: (0, 0)),
        ],
        out_specs=pl.BlockSpec((_M_BLK, _NG), lambda i: (i, 0)),
        out_shape=jax.ShapeDtypeStruct((_B, _NG), jnp.float32),
    )(x, w_ih, b_ih.reshape(1, _NG))


def _gru_scan_body(h0_ref, gi_ref, whh_ref, bhh_ref, y_ref, h_ref):
    t = pl.program_id(0)

    @pl.when(t == 0)
    def _init():
        h_ref[...] = h0_ref[...]

    h = h_ref[...]
    gh = lax.dot_general(h, whh_ref[...], _DN,
                         preferred_element_type=jnp.float32) + bhh_ref[...]
    gi = gi_ref[0]
    r = jax.nn.sigmoid(gi[:, :_NHID] + gh[:, :_NHID])
    z = jax.nn.sigmoid(gi[:, _NHID:2 * _NHID] + gh[:, _NHID:2 * _NHID])
    n = jnp.tanh(gi[:, 2 * _NHID:] + r * gh[:, 2 * _NHID:])
    h_new = (1.0 - z) * n + z * h
    h_ref[...] = h_new
    y_ref[0] = h_new


def _gru_layer(gi, h0, w_hh, b_hh):
    return pl.pallas_call(
        _gru_scan_body,
        grid=(_SEQ,),
        in_specs=[
            pl.BlockSpec((_BATCH, _NHID), lambda t: (0, 0)),
            pl.BlockSpec((1, _BATCH, _NG), lambda t: (t, 0, 0)),
            pl.BlockSpec((_NG, _NHID), lambda t: (0, 0)),
            pl.BlockSpec((1, _NG), lambda t: (0, 0)),
        ],
        out_specs=pl.BlockSpec((1, _BATCH, _NHID), lambda t: (t, 0, 0)),
        out_shape=jax.ShapeDtypeStruct((_SEQ, _BATCH, _NHID), jnp.float32),
        scratch_shapes=[pltpu.VMEM((_BATCH, _NHID), jnp.float32)],
    )(h0, gi, w_hh, b_hh.reshape(1, _NG))


_N_BLK = 2048
_N_GRID = (_NTOKEN + _N_BLK - 1) // _N_BLK


def _decoder(x, dec_w, dec_b):
    return pl.pallas_call(
        _proj_body,
        grid=(_N_GRID, _B // _M_BLK),
        in_specs=[
            pl.BlockSpec((_M_BLK, _NHID), lambda jn, jm: (jm, 0)),
            pl.BlockSpec((_N_BLK, _NHID), lambda jn, jm: (jn, 0)),
            pl.BlockSpec((1, _N_BLK), lambda jn, jm: (0, jn)),
        ],
        out_specs=pl.BlockSpec((_M_BLK, _N_BLK), lambda jn, jm: (jm, jn)),
        out_shape=jax.ShapeDtypeStruct((_B, _NTOKEN), jnp.float32),
    )(x, dec_w, dec_b.reshape(1, _NTOKEN))


def kernel(input, hidden, extra_notes, enc_w, enc_lyr_w, W_ih0, W_hh0, b_ih0,
           b_hh0, W_ih1, W_hh1, b_ih1, b_hh1, dec_w, dec_b):
    half = input.shape[0] // 2
    idx_a = input[:half].reshape(_B)
    idx_b = input[half:].reshape(_B)

    emb, emb2 = _emb_gather(enc_w, idx_a, enc_lyr_w, idx_b)

    gi0 = _input_proj0(emb, emb2, W_ih0, b_ih0)
    y0 = _gru_layer(gi0.reshape(_SEQ, _BATCH, _NG), hidden[0], W_hh0, b_hh0)

    gi1 = _input_proj1(y0.reshape(_B, _NHID), W_ih1, b_ih1)
    y1 = _gru_layer(gi1.reshape(_SEQ, _BATCH, _NG), hidden[1], W_hh1, b_hh1)

    decoded = _decoder(y1.reshape(_B, _NHID), dec_w, dec_b)
    decoded = decoded.reshape(_SEQ, _BATCH, _NTOKEN)
    hidden_out = jnp.stack([y0[-1], y1[-1]], axis=0)
    return decoded, hidden_out
